# baseline (speedup 1.0000x reference)
import functools

import jax
import jax.numpy as jnp
from jax import lax
from jax.experimental import pallas as pl
from jax.experimental.pallas import tpu as pltpu
from jax.experimental.pallas import tpu_sc as plsc

_NW = 32


def _sc_max_gather(Np, per_w, CH, K, C):
    n_chunks = per_w // CH
    G = CH * K
    slices = []
    off = 0
    while off < G:
        sz = min(128, G - off)
        slices.append((off, sz))
        off += sz
    mesh = plsc.VectorSubcoreMesh(core_axis_name="c", subcore_axis_name="s")

    @functools.partial(
        pl.kernel,
        mesh=mesh,
        out_type=jax.ShapeDtypeStruct((Np // 2, C), jnp.int32),
        scratch_types=[
            pltpu.VMEM((G,), jnp.int32),
            pltpu.VMEM((G,), jnp.int32),
            pltpu.VMEM((G, C), jnp.int32),
            pltpu.VMEM((G, C), jnp.int32),
            pltpu.VMEM((CH // 2, C), jnp.int32),
            pltpu.VMEM((CH // 2, C), jnp.int32),
            pltpu.SemaphoreType.DMA,
            pltpu.SemaphoreType.DMA,
            pltpu.SemaphoreType.DMA,
            pltpu.SemaphoreType.DMA,
            pltpu.SemaphoreType.DMA,
            pltpu.SemaphoreType.DMA,
        ],
    )
    def sc_fn(x_hbm, idx_hbm, out_hbm, idx_v0, idx_v1, rows_v0, rows_v1,
              out_v0, out_v1, sem0, sem1, isem0, isem1, osem0, osem1):
        wid = lax.axis_index("s") * 2 + lax.axis_index("c")
        base = wid * per_w
        idx_vs = (idx_v0, idx_v1)
        rows_vs = (rows_v0, rows_v1)
        out_vs = (out_v0, out_v1)
        sems = (sem0, sem1)
        isems = (isem0, isem1)
        osems = (osem0, osem1)

        def idx_copy(ct, b):
            nb = base + ct * CH
            return pltpu.make_async_copy(
                idx_hbm.at[pl.ds(nb * K, G)], idx_vs[b], isems[b])

        def fire_rows(b):
            for (o, s) in slices:
                pltpu.async_copy(
                    x_hbm.at[idx_vs[b].at[pl.ds(o, s)]],
                    rows_vs[b].at[pl.ds(o, s)],
                    sems[b],
                )

        def wait_rows(b):
            for (o, s) in slices:
                pltpu.make_async_copy(
                    x_hbm.at[idx_vs[b].at[pl.ds(o, s)]],
                    rows_vs[b].at[pl.ds(o, s)],
                    sems[b],
                ).wait()

        def out_drain(b):
            pltpu.make_async_copy(
                out_vs[b], out_hbm.at[pl.ds(0, CH // 2)], osems[b]
            ).wait()

        def maybe_when(cond, fn):
            if isinstance(cond, bool):
                if cond:
                    fn()
            else:
                pl.when(cond)(fn)

        def one(ct, b):
            maybe_when(ct + 1 < n_chunks, lambda: idx_copy(ct + 1, b ^ 1).wait())
            maybe_when(ct + 1 < n_chunks, lambda: fire_rows(b ^ 1))

            wait_rows(b)
            maybe_when(ct + 2 < n_chunks, lambda: idx_copy(ct + 2, b).start())

            maybe_when(ct >= 2, lambda: out_drain(b))

            hi_mask = jnp.full((16,), -65536, jnp.int32)
            for ni in range(CH):
                r0 = ni * K
                for cg in range(C // 32):
                    sl = pl.ds(cg * 16, 16)
                    w = rows_vs[b][r0, sl]
                    mlo = lax.bitcast_convert_type(w << 16, jnp.float32)
                    mhi = lax.bitcast_convert_type(w, jnp.float32)
                    for j in range(1, K):
                        w = rows_vs[b][r0 + j, sl]
                        mlo = jnp.maximum(
                            mlo,
                            lax.bitcast_convert_type(w << 16, jnp.float32))
                        mhi = jnp.maximum(
                            mhi, lax.bitcast_convert_type(w, jnp.float32))
                    lo_bits = lax.shift_right_logical(
                        lax.bitcast_convert_type(mlo, jnp.int32), 16)
                    hi_bits = (lax.bitcast_convert_type(mhi, jnp.int32)
                               & hi_mask)
                    out_vs[b][ni // 2,
                              pl.ds((ni % 2) * (C // 2) + cg * 16, 16)] = (
                        lo_bits | hi_bits)

            nb = base + ct * CH
            row0 = pl.multiple_of(nb // 2, 8)
            pltpu.async_copy(out_vs[b], out_hbm.at[pl.ds(row0, CH // 2)],
                             osems[b])

        idx_copy(0, 0).start()
        idx_copy(0, 0).wait()
        fire_rows(0)
        if n_chunks > 1:
            idx_copy(1, 1).start()

        def pair(tp, carry):
            one(2 * tp, 0)
            one(2 * tp + 1, 1)
            return carry

        lax.fori_loop(0, n_chunks // 2, pair, 0)
        for ct in range(2 * (n_chunks // 2), n_chunks):
            one(ct, ct % 2)

        for b in range(min(2, n_chunks)):
            out_drain(b)

    return sc_fn


def _tc_mlp(x_cn, aggr, Wd, W2, b2, NBLK):
    C, N = x_cn.shape
    C_OUT = Wd.shape[0]
    grid = pl.cdiv(N, NBLK)

    def tc_fn(x_ref, a_ref, wd_ref, w2_ref, b_ref, o_ref):
        mm1 = lax.dot_general(
            wd_ref[...], x_ref[...], (((1,), (0,)), ((), ())),
            preferred_element_type=jnp.float32)
        mm2 = lax.dot_general(
            w2_ref[...], a_ref[...].astype(jnp.float32),
            (((1,), (1,)), ((), ())),
            preferred_element_type=jnp.float32)
        o_ref[...] = jnp.maximum(mm1 + mm2 + b_ref[...], 0.0)

    return pl.pallas_call(
        tc_fn,
        grid=(grid,),
        in_specs=[
            pl.BlockSpec((C, NBLK), lambda i: (0, i)),
            pl.BlockSpec((NBLK, C), lambda i: (i, 0)),
            pl.BlockSpec((C_OUT, C), lambda i: (0, 0)),
            pl.BlockSpec((C_OUT, C), lambda i: (0, 0)),
            pl.BlockSpec((C_OUT, 1), lambda i: (0, 0)),
        ],
        out_specs=pl.BlockSpec((C_OUT, NBLK), lambda i: (0, i)),
        out_shape=jax.ShapeDtypeStruct((C_OUT, N), jnp.float32),
    )(x_cn, aggr, Wd, W2, b2)


def kernel(x, edge_index, W, b):
    B, C, N, _ = x.shape
    K = edge_index.shape[-1]
    C_OUT = W.shape[0]

    x_cn = x[0, :, :, 0]
    x_bf = jnp.transpose(x_cn).astype(jnp.bfloat16)
    x_pk = jnp.pad(
        lax.bitcast_convert_type(
            jnp.reshape(x_bf, (N, C // 2, 2)), jnp.int32),
        ((0, 0), (0, C // 2)))
    idx = edge_index[0, 0]

    per_w = (-(-N // _NW) + 15) // 16 * 16
    Np = per_w * _NW
    idx_flat = jnp.pad(jnp.reshape(idx, (-1,)), (0, Np * K - N * K))

    CH = 32
    while per_w % CH:
        CH -= 8
    aggr_pk = _sc_max_gather(Np, per_w, CH, K, C)(x_pk, idx_flat)
    aggr = jnp.reshape(
        lax.bitcast_convert_type(aggr_pk, jnp.bfloat16), (Np, C))

    W1, W2 = W[:, :C], W[:, C:]
    Wd = W1 - W2
    out = _tc_mlp(x_cn, aggr, Wd, W2, jnp.reshape(b, (C_OUT, 1)), 2048)
    return out[None, :, :, None]


# speedup vs baseline: 10.1707x; 10.1707x over previous
import functools

import jax
import jax.numpy as jnp
from jax import lax
from jax.experimental import pallas as pl
from jax.experimental.pallas import tpu as pltpu
from jax.experimental.pallas import tpu_sc as plsc

_NW = 32


def _sc_max_gather(Np, per_w, CH, K, C):
    n_chunks = per_w // CH
    G = CH * K
    slices = []
    off = 0
    while off < G:
        sz = min(128, G - off)
        slices.append((off, sz))
        off += sz
    mesh = plsc.VectorSubcoreMesh(core_axis_name="c", subcore_axis_name="s")

    @functools.partial(
        pl.kernel,
        mesh=mesh,
        out_type=jax.ShapeDtypeStruct((Np, C), jnp.float32),
        scratch_types=[
            pltpu.VMEM((G,), jnp.int32),
            pltpu.VMEM((G,), jnp.int32),
            pltpu.VMEM((G, C), jnp.float32),
            pltpu.VMEM((G, C), jnp.float32),
            pltpu.VMEM((CH, C), jnp.float32),
            pltpu.VMEM((CH, C), jnp.float32),
            pltpu.SemaphoreType.DMA,
            pltpu.SemaphoreType.DMA,
            pltpu.SemaphoreType.DMA,
            pltpu.SemaphoreType.DMA,
            pltpu.SemaphoreType.DMA,
            pltpu.SemaphoreType.DMA,
        ],
    )
    def sc_fn(x_hbm, idx_hbm, out_hbm, idx_v0, idx_v1, rows_v0, rows_v1,
              out_v0, out_v1, sem0, sem1, isem0, isem1, osem0, osem1):
        wid = lax.axis_index("s") * 2 + lax.axis_index("c")
        base = wid * per_w
        idx_vs = (idx_v0, idx_v1)
        rows_vs = (rows_v0, rows_v1)
        out_vs = (out_v0, out_v1)
        sems = (sem0, sem1)
        isems = (isem0, isem1)
        osems = (osem0, osem1)

        def idx_copy(ct, b):
            nb = base + ct * CH
            return pltpu.make_async_copy(
                idx_hbm.at[pl.ds(nb * K, G)], idx_vs[b], isems[b])

        def fire_rows(b):
            for (o, s) in slices:
                pltpu.async_copy(
                    x_hbm.at[idx_vs[b].at[pl.ds(o, s)]],
                    rows_vs[b].at[pl.ds(o, s)],
                    sems[b],
                )

        def wait_rows(b):
            for (o, s) in slices:
                pltpu.make_async_copy(
                    x_hbm.at[idx_vs[b].at[pl.ds(o, s)]],
                    rows_vs[b].at[pl.ds(o, s)],
                    sems[b],
                ).wait()

        def out_drain(b):
            pltpu.make_async_copy(
                out_vs[b], out_hbm.at[pl.ds(0, CH)], osems[b]
            ).wait()

        def maybe_when(cond, fn):
            if isinstance(cond, bool):
                if cond:
                    fn()
            else:
                pl.when(cond)(fn)

        def one(ct, b):
            maybe_when(ct + 1 < n_chunks, lambda: idx_copy(ct + 1, b ^ 1).wait())
            maybe_when(ct + 1 < n_chunks, lambda: fire_rows(b ^ 1))

            wait_rows(b)
            maybe_when(ct + 2 < n_chunks, lambda: idx_copy(ct + 2, b).start())

            maybe_when(ct >= 2, lambda: out_drain(b))

            def node_body(ni, carry):
                r0 = ni * K
                for cg in range(C // 16):
                    sl = pl.ds(cg * 16, 16)
                    m = rows_vs[b][r0, sl]
                    for j in range(1, K):
                        m = jnp.maximum(m, rows_vs[b][r0 + j, sl])
                    out_vs[b][ni, sl] = m
                return carry

            lax.fori_loop(0, CH, node_body, 0)

            nb = base + ct * CH
            row0 = pl.multiple_of(nb, 8)
            pltpu.async_copy(out_vs[b], out_hbm.at[pl.ds(row0, CH)],
                             osems[b])

        idx_copy(0, 0).start()
        idx_copy(0, 0).wait()
        fire_rows(0)
        if n_chunks > 1:
            idx_copy(1, 1).start()

        def pair(tp, carry):
            one(2 * tp, 0)
            one(2 * tp + 1, 1)
            return carry

        lax.fori_loop(0, n_chunks // 2, pair, 0)
        for ct in range(2 * (n_chunks // 2), n_chunks):
            one(ct, ct % 2)

        for b in range(min(2, n_chunks)):
            out_drain(b)

    return sc_fn


def _tc_mlp(x_cn, aggr, Wd, W2, b2, NBLK):
    C, N = x_cn.shape
    C_OUT = Wd.shape[0]
    grid = pl.cdiv(N, NBLK)

    def tc_fn(x_ref, a_ref, wd_ref, w2_ref, b_ref, o_ref):
        mm1 = lax.dot_general(
            wd_ref[...], x_ref[...], (((1,), (0,)), ((), ())),
            preferred_element_type=jnp.float32)
        mm2 = lax.dot_general(
            w2_ref[...], a_ref[...], (((1,), (1,)), ((), ())),
            preferred_element_type=jnp.float32)
        o_ref[...] = jnp.maximum(mm1 + mm2 + b_ref[...], 0.0)

    return pl.pallas_call(
        tc_fn,
        grid=(grid,),
        in_specs=[
            pl.BlockSpec((C, NBLK), lambda i: (0, i)),
            pl.BlockSpec((NBLK, C), lambda i: (i, 0)),
            pl.BlockSpec((C_OUT, C), lambda i: (0, 0)),
            pl.BlockSpec((C_OUT, C), lambda i: (0, 0)),
            pl.BlockSpec((C_OUT, 1), lambda i: (0, 0)),
        ],
        out_specs=pl.BlockSpec((C_OUT, NBLK), lambda i: (0, i)),
        out_shape=jax.ShapeDtypeStruct((C_OUT, N), jnp.float32),
    )(x_cn, aggr, Wd, W2, b2)


def kernel(x, edge_index, W, b):
    B, C, N, _ = x.shape
    K = edge_index.shape[-1]
    C_OUT = W.shape[0]

    x_cn = x[0, :, :, 0]
    x_nc = jnp.transpose(x_cn)
    idx = edge_index[0, 0]

    per_w = (-(-N // _NW) + 15) // 16 * 16
    Np = per_w * _NW
    idx_flat = jnp.pad(jnp.reshape(idx, (-1,)), (0, Np * K - N * K))

    CH = 32
    while per_w % CH:
        CH -= 8
    aggr = _sc_max_gather(Np, per_w, CH, K, C)(x_nc, idx_flat)

    W1, W2 = W[:, :C], W[:, C:]
    Wd = W1 - W2
    out = _tc_mlp(x_cn, aggr, Wd, W2, jnp.reshape(b, (C_OUT, 1)), 2048)
    return out[None, :, :, None]


# speedup vs baseline: 10.7168x; 1.0537x over previous
import functools

import jax
import jax.numpy as jnp
from jax import lax
from jax.experimental import pallas as pl
from jax.experimental.pallas import tpu as pltpu
from jax.experimental.pallas import tpu_sc as plsc

_NW = 32


def _sc_max_gather(Np, per_w, CH, K, C):
    n_chunks = per_w // CH
    G = CH * K
    slices = []
    off = 0
    while off < G:
        sz = min(128, G - off)
        slices.append((off, sz))
        off += sz
    mesh = plsc.VectorSubcoreMesh(core_axis_name="c", subcore_axis_name="s")

    @functools.partial(
        pl.kernel,
        mesh=mesh,
        out_type=jax.ShapeDtypeStruct((Np, C), jnp.float32),
        scratch_types=[
            pltpu.VMEM((G,), jnp.int32),
            pltpu.VMEM((G,), jnp.int32),
            pltpu.VMEM((G, C), jnp.float32),
            pltpu.VMEM((G, C), jnp.float32),
            pltpu.VMEM((CH, C), jnp.float32),
            pltpu.VMEM((CH, C), jnp.float32),
            pltpu.SemaphoreType.DMA,
            pltpu.SemaphoreType.DMA,
            pltpu.SemaphoreType.DMA,
            pltpu.SemaphoreType.DMA,
            pltpu.SemaphoreType.DMA,
            pltpu.SemaphoreType.DMA,
        ],
    )
    def sc_fn(x_hbm, idx_hbm, out_hbm, idx_v0, idx_v1, rows_v0, rows_v1,
              out_v0, out_v1, sem0, sem1, isem0, isem1, osem0, osem1):
        wid = lax.axis_index("s") * 2 + lax.axis_index("c")
        base = wid * per_w
        idx_vs = (idx_v0, idx_v1)
        rows_vs = (rows_v0, rows_v1)
        out_vs = (out_v0, out_v1)
        sems = (sem0, sem1)
        isems = (isem0, isem1)
        osems = (osem0, osem1)

        def idx_copy(ct, b):
            nb = base + ct * CH
            return pltpu.make_async_copy(
                idx_hbm.at[pl.ds(nb * K, G)], idx_vs[b], isems[b])

        def fire_rows(b):
            for (o, s) in slices:
                pltpu.async_copy(
                    x_hbm.at[idx_vs[b].at[pl.ds(o, s)]],
                    rows_vs[b].at[pl.ds(o, s)],
                    sems[b],
                )

        def wait_rows(b):
            for (o, s) in slices:
                pltpu.make_async_copy(
                    x_hbm.at[idx_vs[b].at[pl.ds(o, s)]],
                    rows_vs[b].at[pl.ds(o, s)],
                    sems[b],
                ).wait()

        def out_drain(b):
            pltpu.make_async_copy(
                out_vs[b], out_hbm.at[pl.ds(0, CH)], osems[b]
            ).wait()

        def maybe_when(cond, fn):
            if isinstance(cond, bool):
                if cond:
                    fn()
            else:
                pl.when(cond)(fn)

        def one(ct, b):
            maybe_when(ct + 1 < n_chunks, lambda: idx_copy(ct + 1, b ^ 1).wait())
            maybe_when(ct + 1 < n_chunks, lambda: fire_rows(b ^ 1))

            wait_rows(b)
            maybe_when(ct + 2 < n_chunks, lambda: idx_copy(ct + 2, b).start())

            maybe_when(ct >= 2, lambda: out_drain(b))

            @plsc.parallel_loop(0, CH, unroll=2)
            def node_body(ni):
                r0 = ni * K
                for cg in range(C // 16):
                    sl = pl.ds(cg * 16, 16)
                    w = [rows_vs[b][r0 + j, sl] for j in range(K)]
                    while len(w) > 1:
                        w = [jnp.maximum(w[i], w[i + 1])
                             for i in range(0, len(w) - 1, 2)] + (
                                 [w[-1]] if len(w) % 2 else [])
                    out_vs[b][ni, sl] = w[0]

            nb = base + ct * CH
            row0 = pl.multiple_of(nb, 8)
            pltpu.async_copy(out_vs[b], out_hbm.at[pl.ds(row0, CH)],
                             osems[b])

        idx_copy(0, 0).start()
        idx_copy(0, 0).wait()
        fire_rows(0)
        if n_chunks > 1:
            idx_copy(1, 1).start()

        def pair(tp, carry):
            one(2 * tp, 0)
            one(2 * tp + 1, 1)
            return carry

        lax.fori_loop(0, n_chunks // 2, pair, 0)
        for ct in range(2 * (n_chunks // 2), n_chunks):
            one(ct, ct % 2)

        for b in range(min(2, n_chunks)):
            out_drain(b)

    return sc_fn


def _tc_mlp(x_cn, aggr, Wd, W2, b2, NBLK):
    C, N = x_cn.shape
    C_OUT = Wd.shape[0]
    grid = pl.cdiv(N, NBLK)

    def tc_fn(x_ref, a_ref, wd_ref, w2_ref, b_ref, o_ref):
        mm1 = lax.dot_general(
            wd_ref[...], x_ref[...], (((1,), (0,)), ((), ())),
            preferred_element_type=jnp.float32)
        mm2 = lax.dot_general(
            w2_ref[...], a_ref[...], (((1,), (1,)), ((), ())),
            preferred_element_type=jnp.float32)
        o_ref[...] = jnp.maximum(mm1 + mm2 + b_ref[...], 0.0)

    return pl.pallas_call(
        tc_fn,
        grid=(grid,),
        in_specs=[
            pl.BlockSpec((C, NBLK), lambda i: (0, i)),
            pl.BlockSpec((NBLK, C), lambda i: (i, 0)),
            pl.BlockSpec((C_OUT, C), lambda i: (0, 0)),
            pl.BlockSpec((C_OUT, C), lambda i: (0, 0)),
            pl.BlockSpec((C_OUT, 1), lambda i: (0, 0)),
        ],
        out_specs=pl.BlockSpec((C_OUT, NBLK), lambda i: (0, i)),
        out_shape=jax.ShapeDtypeStruct((C_OUT, N), jnp.float32),
    )(x_cn, aggr, Wd, W2, b2)


def kernel(x, edge_index, W, b):
    B, C, N, _ = x.shape
    K = edge_index.shape[-1]
    C_OUT = W.shape[0]

    x_cn = x[0, :, :, 0]
    x_nc = jnp.transpose(x_cn)
    idx = edge_index[0, 0]

    per_w = (-(-N // _NW) + 15) // 16 * 16
    Np = per_w * _NW
    idx_flat = jnp.pad(jnp.reshape(idx, (-1,)), (0, Np * K - N * K))

    CH = 32
    while per_w % CH:
        CH -= 8
    aggr = _sc_max_gather(Np, per_w, CH, K, C)(x_nc, idx_flat)

    W1, W2 = W[:, :C], W[:, C:]
    Wd = W1 - W2
    out = _tc_mlp(x_cn, aggr, Wd, W2, jnp.reshape(b, (C_OUT, 1)), 2048)
    return out[None, :, :, None]
